# Initial kernel scaffold; baseline (speedup 1.0000x reference)
import jax
import jax.numpy as jnp
from jax.experimental import pallas as pl


def kernel(x, edge_index, edge_type, ppi_list, idx, W, b, linear_W, linear_b, fc_W, fc_b):
    raise NotImplementedError("write your pallas kernel here")


# speedup vs baseline: 9.1514x; 9.1514x over previous
import functools

import jax
import jax.numpy as jnp
from jax import lax
from jax.experimental import pallas as pl
from jax.experimental.pallas import tpu as pltpu
from jax.experimental.pallas import tpu_sc as plsc

N = 10000
E = 160000
D = 256
R = 8
S = R * N
OUTD = 7
NPAIRS = 100000
BATCH = 16384

NC = 2
NS = 16
LANES = 16

S_PAD = 81920
E_PAD = 163840
TRASH = S_PAD - 1
EPT_A = E_PAD // (NC * NS)
NBA = EPT_A // 128
EPT_D = E_PAD // NS
CH = 4096
NPASS = S_PAD // (CH * NC)
STRIPE = CH // NS
SPT = S_PAD // NS

_mesh = plsc.VectorSubcoreMesh(core_axis_name="c", subcore_axis_name="s")
_sc_params = pltpu.CompilerParams(needs_layout_passes=False)


@functools.partial(
    pl.kernel,
    out_type=[jax.ShapeDtypeStruct((NC * NS, S_PAD), jnp.float32),
              jax.ShapeDtypeStruct((NC * NS, S_PAD), jnp.float32)],
    mesh=_mesh,
    compiler_params=_sc_params,
    scratch_types=[
        pltpu.VMEM((EPT_A,), jnp.int32),
        pltpu.VMEM((S_PAD,), jnp.float32),
    ],
)
def _sc_degrees(oslot_hbm, islot_hbm, dego_hbm, degi_hbm, slots_v, hist_v):
    c = lax.axis_index("c")
    s = lax.axis_index("s")
    w = c * NS + s
    ones = jnp.ones((LANES,), jnp.float32)
    zf = jnp.zeros((LANES,), jnp.float32)

    for pi in range(2):
        slot_hbm = (oslot_hbm, islot_hbm)[pi]
        out_hbm = (dego_hbm, degi_hbm)[pi]

        def zero_body(i, _):
            hist_v[pl.ds(i * LANES, LANES)] = zf
            return 0
        lax.fori_loop(0, S_PAD // LANES, zero_body, 0)

        pltpu.sync_copy(slot_hbm.at[pl.ds(w * EPT_A, EPT_A)], slots_v)

        def hg_body(g, _):
            vec = slots_v[pl.ds(g * LANES, LANES)]
            plsc.addupdate_scatter(hist_v, [vec], ones)
            return 0
        lax.fori_loop(0, EPT_A // LANES, hg_body, 0)

        pltpu.sync_copy(hist_v, out_hbm.at[w])


@functools.partial(
    pl.kernel,
    out_type=jax.ShapeDtypeStruct((2 * S_PAD, 128), jnp.float32),
    mesh=_mesh,
    compiler_params=_sc_params,
    scratch_types=[
        pltpu.VMEM((EPT_D,), jnp.int32),
        pltpu.VMEM((EPT_D,), jnp.int32),
        pltpu.VMEM((EPT_D + LANES,), jnp.int32),
        pltpu.VMEM((EPT_D + LANES,), jnp.int32),
        pltpu.VMEM((LANES, 128), jnp.float32),
        pltpu.VMEM((LANES, 128), jnp.float32),
        pltpu.VMEM((LANES, 128), jnp.float32),
        pltpu.VMEM((LANES, 128), jnp.float32),
        pltpu.VMEM((64, 128), jnp.float32),
        pltpu.VMEM_SHARED((2 * (CH + LANES), 128), jnp.float32),
        pltpu.SemaphoreType.DMA,
        pltpu.SemaphoreType.DMA,
        pltpu.SemaphoreType.DMA,
        pltpu.SemaphoreType.DMA,
    ],
)
def _sc_edge_pass(hs_hbm, islot_hbm, oslot_hbm, agg_hbm,
                  isv, osv, cs, cg, buf0a, buf0b, buf1a, buf1b, zbuf,
                  chunk_sh, sem0a, sem0b, sem1a, sem1b):
    c = lax.axis_index("c")
    s = lax.axis_index("s")
    sems = ((sem0a, sem0b), (sem1a, sem1b))
    bufs = ((buf0a, buf0b), (buf1a, buf1b))
    zf = jnp.zeros((LANES,), jnp.float32)

    pltpu.sync_copy(islot_hbm.at[pl.ds(s * EPT_D, EPT_D)], isv)
    pltpu.sync_copy(oslot_hbm.at[pl.ds(s * EPT_D, EPT_D)], osv)

    def zb_body(i, _):
        for j in range(128 // LANES):
            zbuf[i, pl.ds(j * LANES, LANES)] = zf
        return 0
    lax.fori_loop(0, 64, zb_body, 0)

    for p in range(NPASS):
        lo = (p * NC + c) * CH
        hi = lo + CH
        for z in range(2 * STRIPE // 64):
            pltpu.sync_copy(zbuf,
                            chunk_sh.at[pl.ds(2 * s * STRIPE + z * 64, 64)])
        plsc.subcore_barrier()

        def cmp_body(g, off):
            iv = isv[pl.ds(g * LANES, LANES)]
            ov = osv[pl.ds(g * LANES, LANES)]
            m = (iv >= lo) & (iv < hi)
            plsc.store_compressed(cs.at[pl.ds(off, LANES)], iv - lo, mask=m)
            plsc.store_compressed(cg.at[pl.ds(off, LANES)], ov, mask=m)
            cnt = plsc.all_reduce_population_count(m)
            return off + cnt[0]
        m_edges = lax.fori_loop(0, EPT_D // LANES, cmp_body, jnp.int32(0))

        cs[pl.ds(m_edges, LANES)] = jnp.full((LANES,), CH, jnp.int32)
        cg[pl.ds(m_edges, LANES)] = jnp.zeros((LANES,), jnp.int32)
        ng = (m_edges + LANES - 1) // LANES

        def gs_body(g2, _):
            for k in range(2):
                g = g2 * 2 + k

                @pl.when(g < ng)
                def _():
                    gi = cg[pl.ds(g * LANES, LANES)] * 2
                    pltpu.async_copy(hs_hbm.at[gi], bufs[k][0], sems[k][0])
                    pltpu.async_copy(hs_hbm.at[gi + 1], bufs[k][1], sems[k][1])
            for k in range(2):
                g = g2 * 2 + k

                @pl.when(g < ng)
                def _():
                    si = cs[pl.ds(g * LANES, LANES)] * 2
                    for h in range(2):
                        pltpu.make_async_copy(hs_hbm.at[pl.ds(0, LANES)],
                                              bufs[k][h], sems[k][h]).wait()
                        pltpu.sync_copy(bufs[k][h], chunk_sh.at[si + h],
                                        add=True)
            return 0
        lax.fori_loop(0, (ng + 1) // 2, gs_body, 0)
        plsc.subcore_barrier()

        pltpu.sync_copy(chunk_sh.at[pl.ds(2 * s * STRIPE, 2 * STRIPE)],
                        agg_hbm.at[pl.ds(2 * (lo + s * STRIPE), 2 * STRIPE)])


PPT = BATCH // (NC * NS)
QS = NPAIRS // 4


@functools.partial(
    pl.kernel,
    out_type=jax.ShapeDtypeStruct((BATCH, D), jnp.float32),
    mesh=_mesh,
    compiler_params=_sc_params,
    scratch_types=[
        pltpu.VMEM((PPT,), jnp.int32),
        pltpu.VMEM((PPT,), jnp.int32),
        pltpu.VMEM((PPT,), jnp.int32),
        pltpu.VMEM((QS,), jnp.int32),
        pltpu.VMEM((128, D), jnp.float32),
        pltpu.VMEM((128, D), jnp.float32),
        pltpu.SemaphoreType.DMA,
        pltpu.SemaphoreType.DMA,
    ],
)
def _sc_pairs(hx_hbm, ppia_hbm, ppib_hbm, idx_hbm, p_hbm,
              idxv, aidx, bidx, part, abuf, bbuf, sema, semb):
    c = lax.axis_index("c")
    s = lax.axis_index("s")
    base = (c * NS + s) * PPT

    pltpu.sync_copy(idx_hbm.at[pl.ds(base, PPT)], idxv)

    for col in range(2):
        src_hbm = (ppia_hbm, ppib_hbm)[col]
        dstbuf = (aidx, bidx)[col]
        for q in range(4):
            pltpu.sync_copy(src_hbm.at[pl.ds(q * QS, QS)], part)

            def gq_body(g, _):
                iv = idxv[pl.ds(g * LANES, LANES)]
                m = (iv >= q * QS) & (iv < (q + 1) * QS)
                liv = jnp.clip(iv - q * QS, 0, QS - 1)
                got = plsc.load_gather(part, [liv])
                old = dstbuf[pl.ds(g * LANES, LANES)]
                dstbuf[pl.ds(g * LANES, LANES)] = jnp.where(m, got, old)
                return 0
            lax.fori_loop(0, PPT // LANES, gq_body, 0)

    for blk in range(PPT // 128):
        pltpu.async_copy(hx_hbm.at[aidx.at[pl.ds(blk * 128, 128)]],
                         abuf, sema)
        pltpu.async_copy(hx_hbm.at[bidx.at[pl.ds(blk * 128, 128)]],
                         bbuf, semb)
        pltpu.make_async_copy(hx_hbm.at[pl.ds(0, 128)], abuf, sema).wait()
        pltpu.make_async_copy(hx_hbm.at[pl.ds(0, 128)], bbuf, semb).wait()

        def mul_body(i, _):
            for j in range(D // LANES):
                abuf[i, pl.ds(j * LANES, LANES)] = (
                    abuf[i, pl.ds(j * LANES, LANES)]
                    * bbuf[i, pl.ds(j * LANES, LANES)])
            return 0
        lax.fori_loop(0, 128, mul_body, 0)
        pltpu.sync_copy(abuf, p_hbm.at[pl.ds(base + blk * 128, 128)])


def _norm_body(do_ref, di_ref, ns_ref, nd_ref):
    t = jnp.maximum(jnp.sum(do_ref[...], axis=0), 1.0)
    ns_ref[...] = lax.rsqrt(t).reshape(64, 128)
    t2 = jnp.maximum(jnp.sum(di_ref[...], axis=0), 1.0)
    nd_ref[...] = lax.rsqrt(t2).reshape(64, 128)


def _tc_norms(dego, degi):
    ns, nd = pl.pallas_call(
        _norm_body,
        grid=(10,),
        in_specs=[pl.BlockSpec((NC * NS, 8192), lambda i: (0, i)),
                  pl.BlockSpec((NC * NS, 8192), lambda i: (0, i))],
        out_specs=[pl.BlockSpec((64, 128), lambda i: (i, 0)),
                   pl.BlockSpec((64, 128), lambda i: (i, 0))],
        out_shape=[jax.ShapeDtypeStruct((640, 128), jnp.float32),
                   jax.ShapeDtypeStruct((640, 128), jnp.float32)],
    )(dego, degi)
    return ns.reshape(S_PAD)[:S], nd.reshape(S_PAD)[:S]


def _prescale_body(h_ref, ns_ref, hs_ref):
    hs_ref[0] = h_ref[...] * ns_ref[...]


def _tc_prescale(h, ns_col):
    return pl.pallas_call(
        _prescale_body,
        grid=(R, 10),
        in_specs=[pl.BlockSpec((1000, D), lambda r, i: (i, 0)),
                  pl.BlockSpec((1000, 1), lambda r, i: (r * 10 + i, 0))],
        out_specs=pl.BlockSpec((1, 1000, D), lambda r, i: (r, i, 0)),
        out_shape=jax.ShapeDtypeStruct((R, N, D), jnp.float32),
    )(h, ns_col)


def _layer_body(agg_ref, nd_ref, w_ref, b_ref, out_ref, acc_ref):
    r = pl.program_id(1)
    a = agg_ref[0] * nd_ref[...]
    part = jnp.dot(a, w_ref[0], preferred_element_type=jnp.float32)

    @pl.when(r == 0)
    def _():
        acc_ref[...] = part

    @pl.when(r > 0)
    def _():
        acc_ref[...] = acc_ref[...] + part

    @pl.when(r == R - 1)
    def _():
        bsum = jnp.sum(b_ref[...], axis=0)
        out_ref[...] = jnp.maximum(acc_ref[...] + bsum[None, :], 0.0)


def _tc_layer(agg3, nd_col, w_l, b_l):
    return pl.pallas_call(
        _layer_body,
        grid=(10, R),
        in_specs=[pl.BlockSpec((1, 1000, D), lambda i, r: (r, i, 0)),
                  pl.BlockSpec((1000, 1), lambda i, r: (r * 10 + i, 0)),
                  pl.BlockSpec((1, D, D), lambda i, r: (r, 0, 0)),
                  pl.BlockSpec((R, D), lambda i, r: (0, 0))],
        out_specs=pl.BlockSpec((1000, D), lambda i, r: (i, 0)),
        out_shape=jax.ShapeDtypeStruct((N, D), jnp.float32),
        scratch_shapes=[pltpu.VMEM((1000, D), jnp.float32)],
    )(agg3, nd_col, w_l, b_l)


def _linear_body(h_ref, w_ref, b_ref, out_ref):
    out_ref[...] = jnp.maximum(
        jnp.dot(h_ref[...], w_ref[...], preferred_element_type=jnp.float32)
        + b_ref[0][None, :], 0.0)


def _tc_linear(h, w, bvec):
    return pl.pallas_call(
        _linear_body,
        grid=(10,),
        in_specs=[pl.BlockSpec((1000, D), lambda i: (i, 0)),
                  pl.BlockSpec((D, D), lambda i: (0, 0)),
                  pl.BlockSpec((1, D), lambda i: (0, 0))],
        out_specs=pl.BlockSpec((1000, D), lambda i: (i, 0)),
        out_shape=jax.ShapeDtypeStruct((N, D), jnp.float32),
    )(h, w, bvec.reshape(1, D))


def _fc_body(p_ref, w_ref, b_ref, out_ref):
    out_ref[...] = (
        jnp.dot(p_ref[...], w_ref[...], preferred_element_type=jnp.float32)
        + b_ref[0][None, :])


def _tc_fc(p, w, bvec):
    return pl.pallas_call(
        _fc_body,
        grid=(8,),
        in_specs=[pl.BlockSpec((2048, D), lambda i: (i, 0)),
                  pl.BlockSpec((D, OUTD), lambda i: (0, 0)),
                  pl.BlockSpec((1, OUTD), lambda i: (0, 0))],
        out_specs=pl.BlockSpec((2048, OUTD), lambda i: (i, 0)),
        out_shape=jax.ShapeDtypeStruct((BATCH, OUTD), jnp.float32),
    )(p, w, bvec.reshape(1, OUTD))


def kernel(x, edge_index, edge_type, ppi_list, idx, W, b,
           linear_W, linear_b, fc_W, fc_b):
    src = edge_index[0]
    dst = edge_index[1]
    pad_t = jnp.full((E_PAD - E,), TRASH, jnp.int32)
    pad_z = jnp.zeros((E_PAD - E,), jnp.int32)
    oslot = jnp.concatenate([edge_type * N + src, pad_z])
    islot = jnp.concatenate([edge_type * N + dst, pad_t])
    oslot_deg = jnp.concatenate([edge_type * N + src, pad_t])

    dego, degi = _sc_degrees(oslot_deg, islot)
    nsf, ndf = _tc_norms(dego, degi)
    ns_col = nsf.reshape(S, 1)
    nd_col = ndf.reshape(S, 1)

    h = x
    for l in range(2):
        hs = _tc_prescale(h, ns_col).reshape(2 * S, 128)
        agg = _sc_edge_pass(hs, islot, oslot)
        agg = agg.reshape(S_PAD, D)[:S]
        h = _tc_layer(agg.reshape(R, N, D), nd_col, W[l], b[l])

    hx = _tc_linear(h, linear_W, linear_b)
    ppia = ppi_list[:, 0] + 0
    ppib = ppi_list[:, 1] + 0
    p = _sc_pairs(hx, ppia, ppib, idx)
    return _tc_fc(p, fc_W, fc_b)
